# baseline (speedup 1.0000x reference)
import math

import jax
import jax.numpy as jnp
import numpy as np
from jax.experimental import pallas as pl
from jax.experimental.pallas import tpu as pltpu

_K = 5
_BIG = 1e30

_HI = jax.lax.Precision.HIGHEST

_NOISE_CACHE = {}


def _threefry2x32(k0, k1, x0, x1):
    def rol(x, d):
        return ((x << np.uint32(d)) | (x >> np.uint32(32 - d))).astype(np.uint32)

    ks2 = np.uint32(k0 ^ k1 ^ np.uint32(0x1BD11BDA))
    ks = [np.uint32(k0), np.uint32(k1), ks2]
    rot = ((13, 15, 26, 6), (17, 29, 16, 24))
    x0 = (x0 + ks[0]).astype(np.uint32)
    x1 = (x1 + ks[1]).astype(np.uint32)
    for i in range(5):
        for r in rot[i % 2]:
            x0 = (x0 + x1).astype(np.uint32)
            x1 = rol(x1, r) ^ x0
        x0 = (x0 + ks[(i + 1) % 3]).astype(np.uint32)
        x1 = (x1 + ks[(i + 2) % 3] + np.uint32(i + 1)).astype(np.uint32)
    return x0, x1


def _noise_const(bsz, n):
    key = (bsz, n)
    if key not in _NOISE_CACHE:
        cnt = bsz * n
        counts = np.arange(cnt, dtype=np.uint32)
        y0, y1 = _threefry2x32(np.uint32(0), np.uint32(42),
                               np.zeros(cnt, dtype=np.uint32), counts)
        bits = y0 ^ y1
        flt = ((bits >> np.uint32(9)) | np.uint32(0x3F800000)).view(np.float32)
        uni = np.maximum(np.float32(0.0), flt - np.float32(1.0))
        noise = (uni * np.float32(1e-06)).astype(np.float32)
        _NOISE_CACHE[key] = noise.reshape(bsz, 1, n)
    return _NOISE_CACHE[key]


def _dpc_kernel(n, cn, cnp, x_ref, noise_ref, w_ref, b_ref, out_ref):
    f32 = jnp.float32
    i32 = jnp.int32
    x = x_ref[0]
    c = x.shape[1]
    rsc = f32(math.sqrt(c))

    ri = jax.lax.broadcasted_iota(i32, (n, n), 0)

    x2_col = jnp.sum(x * x, axis=1, keepdims=True)
    xb = x.astype(jnp.bfloat16)
    g = jax.lax.dot_general(xb, xb, (((1,), (1,)), ((), ())),
                            preferred_element_type=f32)
    x2_row = jnp.transpose(x2_col)
    d2 = jnp.maximum(x2_col + x2_row - 2.0 * g, 0.0)

    dw = d2
    acc = jnp.zeros((1, n), dtype=f32)
    for _ in range(_K):
        m = jnp.min(dw, axis=0, keepdims=True)
        first = jnp.min(jnp.where(dw == m, ri, n), axis=0, keepdims=True)
        dw = jnp.where(ri == first, _BIG, dw)
        dn = jnp.sqrt(m) / rsc
        acc = acc + dn * dn
    dens_row = jnp.exp(-acc / f32(_K)) + noise_ref[0]
    dens_col = jnp.transpose(dens_row)

    d2max0 = jnp.max(d2, axis=0, keepdims=True)
    d2max = jnp.max(d2max0, axis=1, keepdims=True)
    masked = jnp.where(dens_col > dens_row, d2, d2max)
    dmin_row = jnp.sqrt(jnp.min(masked, axis=0, keepdims=True)) / rsc
    score = dmin_row * dens_row

    li = jax.lax.broadcasted_iota(i32, (1, n), 1)
    r16 = jax.lax.broadcasted_iota(i32, (cnp, n), 0)
    onehot = jnp.zeros((cnp, n), dtype=f32)
    centerval = jnp.zeros((1, n), dtype=i32)
    iscenter = jnp.zeros((1, n), dtype=jnp.bool_)
    score_w = score
    for cc in range(cn):
        v = jnp.max(score_w, axis=1, keepdims=True)
        fi = jnp.min(jnp.where(score_w == v, li, n), axis=1, keepdims=True)
        sel = li == fi
        onehot = onehot + jnp.where((r16 == cc) & sel, 1.0, 0.0)
        centerval = jnp.where(sel, cc, centerval)
        iscenter = iscenter | sel
        score_w = jnp.where(sel, -_BIG, score_w)

    dm2 = jax.lax.dot_general(onehot, d2, (((1,), (0,)), ((), ())),
                              preferred_element_type=f32, precision=_HI)
    dm = jnp.sqrt(dm2) / rsc

    best = jnp.full((1, n), _BIG, dtype=f32)
    barg = jnp.zeros((1, n), dtype=i32)
    for cc in range(cn):
        row = jax.lax.slice(dm, (cc, 0), (cc + 1, n))
        upd = row < best
        best = jnp.where(upd, row, best)
        barg = jnp.where(upd, cc, barg)
    idx = jnp.where(iscenter, centerval, barg)

    wb = w_ref[:, :].astype(jnp.bfloat16)
    tscore = jax.lax.dot_general(wb, xb, (((1,), (1,)), ((), ())),
                                 preferred_element_type=f32)
    tw = jnp.exp(tscore + b_ref[:, :])
    a0 = (r16 == idx).astype(f32)
    p = a0 * tw
    allw = jnp.sum(p, axis=1, keepdims=True) + 1e-06
    a = p / allw
    merged = jax.lax.dot_general(a, x, (((1,), (0,)), ((), ())),
                                 preferred_element_type=f32, precision=_HI)
    out_ref[0] = jax.lax.slice(merged, (0, 0), (cn, x.shape[1]))


def kernel(patch_token, anomaly_map, W, b):
    del anomaly_map
    bsz, n, c = patch_token.shape
    cn = max(int(math.ceil(n * 0.01)), 1)
    cnp = ((cn + 7) // 8) * 8
    noise3 = jnp.asarray(_noise_const(bsz, n))
    b2 = jnp.reshape(b, (1, 1)).astype(jnp.float32)

    def body(x_ref, noise_ref, w_ref, b_ref, out_ref):
        _dpc_kernel(n, cn, cnp, x_ref, noise_ref, w_ref, b_ref, out_ref)

    def call(x, nz, w, bb):
        return pl.pallas_call(
            body,
            grid=(x.shape[0],),
            in_specs=[
                pl.BlockSpec((1, n, c), lambda i: (i, 0, 0)),
                pl.BlockSpec((1, 1, n), lambda i: (i, 0, 0)),
                pl.BlockSpec((1, c), lambda i: (0, 0)),
                pl.BlockSpec((1, 1), lambda i: (0, 0)),
            ],
            out_specs=pl.BlockSpec((1, cn, c), lambda i: (i, 0, 0)),
            out_shape=jax.ShapeDtypeStruct((x.shape[0], cn, c), jnp.float32),
            compiler_params=pltpu.CompilerParams(
                dimension_semantics=("arbitrary",),
            ),
        )(x, nz, w, bb)

    devs = jax.devices()
    nd = 2 if (len(devs) >= 2 and bsz % 2 == 0) else 1
    if nd == 1:
        return call(patch_token, noise3, W, b2)
    P = jax.sharding.PartitionSpec
    mesh = jax.sharding.Mesh(np.array(devs[:nd]), ("d",))
    fn = jax.shard_map(
        call,
        mesh=mesh,
        in_specs=(P("d", None, None), P("d", None, None),
                  P(None, None), P(None, None)),
        out_specs=P("d", None, None),
        check_vma=False,
    )
    return fn(patch_token, noise3, W, b2)


# speedup vs baseline: 4.0341x; 4.0341x over previous
import math

import jax
import jax.numpy as jnp
import numpy as np
from jax.experimental import pallas as pl
from jax.experimental.pallas import tpu as pltpu

_K = 5
_BIG = 1e30

_HI = jax.lax.Precision.HIGHEST

_NOISE_CACHE = {}


def _threefry2x32(k0, k1, x0, x1):
    def rol(x, d):
        return ((x << np.uint32(d)) | (x >> np.uint32(32 - d))).astype(np.uint32)

    ks2 = np.uint32(k0 ^ k1 ^ np.uint32(0x1BD11BDA))
    ks = [np.uint32(k0), np.uint32(k1), ks2]
    rot = ((13, 15, 26, 6), (17, 29, 16, 24))
    x0 = (x0 + ks[0]).astype(np.uint32)
    x1 = (x1 + ks[1]).astype(np.uint32)
    for i in range(5):
        for r in rot[i % 2]:
            x0 = (x0 + x1).astype(np.uint32)
            x1 = rol(x1, r) ^ x0
        x0 = (x0 + ks[(i + 1) % 3]).astype(np.uint32)
        x1 = (x1 + ks[(i + 2) % 3] + np.uint32(i + 1)).astype(np.uint32)
    return x0, x1


def _noise_const(bsz, n):
    key = (bsz, n)
    if key not in _NOISE_CACHE:
        cnt = bsz * n
        counts = np.arange(cnt, dtype=np.uint32)
        y0, y1 = _threefry2x32(np.uint32(0), np.uint32(42),
                               np.zeros(cnt, dtype=np.uint32), counts)
        bits = y0 ^ y1
        flt = ((bits >> np.uint32(9)) | np.uint32(0x3F800000)).view(np.float32)
        uni = np.maximum(np.float32(0.0), flt - np.float32(1.0))
        noise = (uni * np.float32(1e-06)).astype(np.float32)
        _NOISE_CACHE[key] = noise.reshape(bsz, 1, n)
    return _NOISE_CACHE[key]


def _dpc_one(n, cn, cnp, x, noise_row, w_ref, b_ref):
    f32 = jnp.float32
    i32 = jnp.int32
    c = x.shape[1]
    rsc = f32(math.sqrt(c))

    x2_col = jnp.sum(x * x, axis=1, keepdims=True)
    xb = x.astype(jnp.bfloat16)
    g = jax.lax.dot_general(xb, xb, (((1,), (1,)), ((), ())),
                            preferred_element_type=f32)
    x2_row = jnp.transpose(x2_col)
    d2 = jnp.maximum(x2_col + x2_row - 2.0 * g, 0.0)

    dw = d2
    acc = jnp.zeros((1, n), dtype=f32)
    rem = jnp.full((1, n), f32(_K))
    for _ in range(_K):
        m = jnp.min(dw, axis=0, keepdims=True)
        eq = dw == m
        cnt = jnp.sum(jnp.where(eq, 1.0, 0.0), axis=0, keepdims=True)
        dw = jnp.where(eq, _BIG, dw)
        take = jnp.minimum(cnt, rem)
        rem = rem - take
        dn = jnp.sqrt(m) / rsc
        acc = acc + take * (dn * dn)
    dens_row = jnp.exp(-acc / f32(_K)) + noise_row
    dens_col = jnp.transpose(dens_row)

    d2max0 = jnp.max(d2, axis=0, keepdims=True)
    d2max = jnp.max(d2max0, axis=1, keepdims=True)
    masked = jnp.where(dens_col > dens_row, d2, d2max)
    dmin_row = jnp.sqrt(jnp.min(masked, axis=0, keepdims=True)) / rsc
    score = dmin_row * dens_row

    li = jax.lax.broadcasted_iota(i32, (1, n), 1)
    r16 = jax.lax.broadcasted_iota(i32, (cnp, n), 0)
    onehot = jnp.zeros((cnp, n), dtype=f32)
    centerval = jnp.zeros((1, n), dtype=i32)
    iscenter = jnp.zeros((1, n), dtype=jnp.bool_)
    score_w = score
    for cc in range(cn):
        v = jnp.max(score_w, axis=1, keepdims=True)
        fi = jnp.min(jnp.where(score_w == v, li, n), axis=1, keepdims=True)
        sel = li == fi
        onehot = onehot + jnp.where((r16 == cc) & sel, 1.0, 0.0)
        centerval = jnp.where(sel, cc, centerval)
        iscenter = iscenter | sel
        score_w = jnp.where(sel, -_BIG, score_w)

    dm2 = jax.lax.dot_general(onehot, d2, (((1,), (0,)), ((), ())),
                              preferred_element_type=f32, precision=_HI)
    dm = jnp.sqrt(dm2) / rsc

    best = jnp.full((1, n), _BIG, dtype=f32)
    barg = jnp.zeros((1, n), dtype=i32)
    for cc in range(cn):
        row = jax.lax.slice(dm, (cc, 0), (cc + 1, n))
        upd = row < best
        best = jnp.where(upd, row, best)
        barg = jnp.where(upd, cc, barg)
    idx = jnp.where(iscenter, centerval, barg)

    wb = w_ref[:, :].astype(jnp.bfloat16)
    tscore = jax.lax.dot_general(wb, xb, (((1,), (1,)), ((), ())),
                                 preferred_element_type=f32)
    tw = jnp.exp(tscore + b_ref[:, :])
    a0 = (r16 == idx).astype(f32)
    p = a0 * tw
    allw = jnp.sum(p, axis=1, keepdims=True) + 1e-06
    a = p / allw
    merged = jax.lax.dot_general(a, x, (((1,), (0,)), ((), ())),
                                 preferred_element_type=f32, precision=_HI)
    return jax.lax.slice(merged, (0, 0), (cn, x.shape[1]))


def kernel(patch_token, anomaly_map, W, b):
    del anomaly_map
    bsz, n, c = patch_token.shape
    cn = max(int(math.ceil(n * 0.01)), 1)
    cnp = ((cn + 7) // 8) * 8
    noise3 = jnp.asarray(_noise_const(bsz, n))
    b2 = jnp.reshape(b, (1, 1)).astype(jnp.float32)
    bb = 1

    def body(x_ref, noise_ref, w_ref, b_ref, out_ref):
        res = [_dpc_one(n, cn, cnp, x_ref[i], noise_ref[i, 0], w_ref, b_ref)
               for i in range(bb)]
        out_ref[:] = jnp.stack(res, axis=0)

    def call(x, nz, w, bbias):
        return pl.pallas_call(
            body,
            grid=(x.shape[0] // bb,),
            in_specs=[
                pl.BlockSpec((bb, n, c), lambda i: (i, 0, 0)),
                pl.BlockSpec((bb, 1, n), lambda i: (i, 0, 0)),
                pl.BlockSpec((1, c), lambda i: (0, 0)),
                pl.BlockSpec((1, 1), lambda i: (0, 0)),
            ],
            out_specs=pl.BlockSpec((bb, cn, c), lambda i: (i, 0, 0)),
            out_shape=jax.ShapeDtypeStruct((x.shape[0], cn, c), jnp.float32),
            compiler_params=pltpu.CompilerParams(
                dimension_semantics=("arbitrary",),
            ),
        )(x, nz, w, bbias)

    return call(patch_token, noise3, W, b2)


# speedup vs baseline: 4.4283x; 1.0977x over previous
import math

import jax
import jax.numpy as jnp
import numpy as np
from jax.experimental import pallas as pl
from jax.experimental.pallas import tpu as pltpu

_K = 5
_BIG = 1e30

_HI = jax.lax.Precision.HIGHEST

_NOISE_CACHE = {}


def _threefry2x32(k0, k1, x0, x1):
    def rol(x, d):
        return ((x << np.uint32(d)) | (x >> np.uint32(32 - d))).astype(np.uint32)

    ks2 = np.uint32(k0 ^ k1 ^ np.uint32(0x1BD11BDA))
    ks = [np.uint32(k0), np.uint32(k1), ks2]
    rot = ((13, 15, 26, 6), (17, 29, 16, 24))
    x0 = (x0 + ks[0]).astype(np.uint32)
    x1 = (x1 + ks[1]).astype(np.uint32)
    for i in range(5):
        for r in rot[i % 2]:
            x0 = (x0 + x1).astype(np.uint32)
            x1 = rol(x1, r) ^ x0
        x0 = (x0 + ks[(i + 1) % 3]).astype(np.uint32)
        x1 = (x1 + ks[(i + 2) % 3] + np.uint32(i + 1)).astype(np.uint32)
    return x0, x1


def _noise_const(bsz, n):
    key = (bsz, n)
    if key not in _NOISE_CACHE:
        cnt = bsz * n
        counts = np.arange(cnt, dtype=np.uint32)
        y0, y1 = _threefry2x32(np.uint32(0), np.uint32(42),
                               np.zeros(cnt, dtype=np.uint32), counts)
        bits = y0 ^ y1
        flt = ((bits >> np.uint32(9)) | np.uint32(0x3F800000)).view(np.float32)
        uni = np.maximum(np.float32(0.0), flt - np.float32(1.0))
        noise = (uni * np.float32(1e-06)).astype(np.float32)
        _NOISE_CACHE[key] = noise.reshape(bsz, 1, n)
    return _NOISE_CACHE[key]


def _dpc_one(n, cn, cnp, x, noise_row, w_ref, b_ref):
    f32 = jnp.float32
    i32 = jnp.int32
    c = x.shape[1]
    rsc = f32(math.sqrt(c))

    x2_col = jnp.sum(x * x, axis=1, keepdims=True)
    xb = x.astype(jnp.bfloat16)
    g = jax.lax.dot_general(xb, xb, (((1,), (1,)), ((), ())),
                            preferred_element_type=f32)
    x2_row = jnp.transpose(x2_col)
    d2 = jnp.maximum(x2_col + x2_row - 2.0 * g, 0.0)

    dw = d2
    acc = jnp.zeros((1, n), dtype=f32)
    rem = jnp.full((1, n), f32(_K))
    for _ in range(_K):
        m = jnp.min(dw, axis=0, keepdims=True)
        eq = dw == m
        cnt = jnp.sum(jnp.where(eq, 1.0, 0.0), axis=0, keepdims=True)
        dw = jnp.where(eq, _BIG, dw)
        take = jnp.minimum(cnt, rem)
        rem = rem - take
        dn = jnp.sqrt(m) / rsc
        acc = acc + take * (dn * dn)
    dens_row = jnp.exp(-acc / f32(_K)) + noise_row
    dens_col = jnp.transpose(dens_row)

    d2max0 = jnp.max(d2, axis=0, keepdims=True)
    d2max = jnp.max(d2max0, axis=1, keepdims=True)
    masked = jnp.where(dens_col > dens_row, d2, d2max)
    dmin_row = jnp.sqrt(jnp.min(masked, axis=0, keepdims=True)) / rsc
    score = dmin_row * dens_row

    li = jax.lax.broadcasted_iota(i32, (1, n), 1)
    r16 = jax.lax.broadcasted_iota(i32, (cnp, n), 0)
    r16c = jax.lax.broadcasted_iota(i32, (cnp, 1), 0)
    onehot = jnp.zeros((cnp, n), dtype=f32)
    centerval = jnp.zeros((1, n), dtype=i32)
    iscenter = jnp.zeros((1, n), dtype=jnp.bool_)
    x2_sel = jnp.zeros((cnp, 1), dtype=f32)
    score_w = score
    for cc in range(cn):
        v = jnp.max(score_w, axis=1, keepdims=True)
        fi = jnp.min(jnp.where(score_w == v, li, n), axis=1, keepdims=True)
        sel = li == fi
        onehot = onehot + jnp.where((r16 == cc) & sel, 1.0, 0.0)
        centerval = jnp.where(sel, cc, centerval)
        iscenter = iscenter | sel
        x2c = jnp.sum(jnp.where(sel, x2_row, 0.0), axis=1, keepdims=True)
        x2_sel = jnp.where(r16c == cc, x2c, x2_sel)
        score_w = jnp.where(sel, -_BIG, score_w)

    ohb = onehot.astype(jnp.bfloat16)
    xcb = jax.lax.dot_general(ohb, xb, (((1,), (0,)), ((), ())),
                              preferred_element_type=f32).astype(jnp.bfloat16)
    g_sel = jax.lax.dot_general(xcb, xb, (((1,), (1,)), ((), ())),
                                preferred_element_type=f32)
    dm2 = jnp.maximum(x2_sel + x2_row - 2.0 * g_sel, 0.0)
    dm = jnp.sqrt(dm2) / rsc

    best = jnp.full((1, n), _BIG, dtype=f32)
    barg = jnp.zeros((1, n), dtype=i32)
    for cc in range(cn):
        row = jax.lax.slice(dm, (cc, 0), (cc + 1, n))
        upd = row < best
        best = jnp.where(upd, row, best)
        barg = jnp.where(upd, cc, barg)
    idx = jnp.where(iscenter, centerval, barg)

    wb = w_ref[:, :].astype(jnp.bfloat16)
    tscore = jax.lax.dot_general(wb, xb, (((1,), (1,)), ((), ())),
                                 preferred_element_type=f32)
    tw = jnp.exp(tscore + b_ref[:, :])
    a0 = (r16 == idx).astype(f32)
    p = a0 * tw
    allw = jnp.sum(p, axis=1, keepdims=True) + 1e-06
    a = p / allw
    merged = jax.lax.dot_general(a, x, (((1,), (0,)), ((), ())),
                                 preferred_element_type=f32, precision=_HI)
    return jax.lax.slice(merged, (0, 0), (cn, x.shape[1]))


def kernel(patch_token, anomaly_map, W, b):
    del anomaly_map
    bsz, n, c = patch_token.shape
    cn = max(int(math.ceil(n * 0.01)), 1)
    cnp = ((cn + 7) // 8) * 8
    noise3 = jnp.asarray(_noise_const(bsz, n))
    b2 = jnp.reshape(b, (1, 1)).astype(jnp.float32)
    bb = 1

    def body(x_ref, noise_ref, w_ref, b_ref, out_ref):
        res = [_dpc_one(n, cn, cnp, x_ref[i], noise_ref[i, 0], w_ref, b_ref)
               for i in range(bb)]
        out_ref[:] = jnp.stack(res, axis=0)

    def call(x, nz, w, bbias):
        return pl.pallas_call(
            body,
            grid=(x.shape[0] // bb,),
            in_specs=[
                pl.BlockSpec((bb, n, c), lambda i: (i, 0, 0)),
                pl.BlockSpec((bb, 1, n), lambda i: (i, 0, 0)),
                pl.BlockSpec((1, c), lambda i: (0, 0)),
                pl.BlockSpec((1, 1), lambda i: (0, 0)),
            ],
            out_specs=pl.BlockSpec((bb, cn, c), lambda i: (i, 0, 0)),
            out_shape=jax.ShapeDtypeStruct((x.shape[0], cn, c), jnp.float32),
            compiler_params=pltpu.CompilerParams(
                dimension_semantics=("arbitrary",),
            ),
        )(x, nz, w, bbias)

    return call(patch_token, noise3, W, b2)


# speedup vs baseline: 4.8475x; 1.0947x over previous
import math

import jax
import jax.numpy as jnp
import numpy as np
from jax.experimental import pallas as pl
from jax.experimental.pallas import tpu as pltpu

_K = 5
_BIG = 1e30

_HI = jax.lax.Precision.HIGHEST

_NOISE_CACHE = {}


def _threefry2x32(k0, k1, x0, x1):
    def rol(x, d):
        return ((x << np.uint32(d)) | (x >> np.uint32(32 - d))).astype(np.uint32)

    ks2 = np.uint32(k0 ^ k1 ^ np.uint32(0x1BD11BDA))
    ks = [np.uint32(k0), np.uint32(k1), ks2]
    rot = ((13, 15, 26, 6), (17, 29, 16, 24))
    x0 = (x0 + ks[0]).astype(np.uint32)
    x1 = (x1 + ks[1]).astype(np.uint32)
    for i in range(5):
        for r in rot[i % 2]:
            x0 = (x0 + x1).astype(np.uint32)
            x1 = rol(x1, r) ^ x0
        x0 = (x0 + ks[(i + 1) % 3]).astype(np.uint32)
        x1 = (x1 + ks[(i + 2) % 3] + np.uint32(i + 1)).astype(np.uint32)
    return x0, x1


def _noise_const(bsz, n):
    key = (bsz, n)
    if key not in _NOISE_CACHE:
        cnt = bsz * n
        counts = np.arange(cnt, dtype=np.uint32)
        y0, y1 = _threefry2x32(np.uint32(0), np.uint32(42),
                               np.zeros(cnt, dtype=np.uint32), counts)
        bits = y0 ^ y1
        flt = ((bits >> np.uint32(9)) | np.uint32(0x3F800000)).view(np.float32)
        uni = np.maximum(np.float32(0.0), flt - np.float32(1.0))
        noise = (uni * np.float32(1e-06)).astype(np.float32)
        _NOISE_CACHE[key] = noise.reshape(bsz, 1, n)
    return _NOISE_CACHE[key]


def _dpc_one(n, cn, cnp, x, noise_row, w_ref, b_ref):
    f32 = jnp.float32
    i32 = jnp.int32
    c = x.shape[1]
    rsc = f32(math.sqrt(c))

    x2_col = jnp.sum(x * x, axis=1, keepdims=True)
    xb = x.astype(jnp.bfloat16)
    g = jax.lax.dot_general(xb, xb, (((1,), (1,)), ((), ())),
                            preferred_element_type=f32)
    x2_row = jnp.transpose(x2_col)
    d2 = jnp.maximum(x2_col + x2_row - 2.0 * g, 0.0)

    dw = d2
    acc = jnp.zeros((1, n), dtype=f32)
    rem = jnp.full((1, n), f32(_K))
    for _ in range(_K):
        m = jnp.min(dw, axis=0, keepdims=True)
        eq = dw == m
        cnt = jnp.sum(jnp.where(eq, 1.0, 0.0), axis=0, keepdims=True)
        dw = jnp.where(eq, _BIG, dw)
        take = jnp.minimum(cnt, rem)
        rem = rem - take
        dn = jnp.sqrt(m) / rsc
        acc = acc + take * (dn * dn)
    dens_row = jnp.exp(-acc / f32(_K)) + noise_row
    dens_col = jnp.transpose(dens_row)

    d2max0 = jnp.max(d2, axis=0, keepdims=True)
    d2max = jnp.max(d2max0, axis=1, keepdims=True)
    masked = jnp.where(dens_col > dens_row, d2, d2max)
    dmin_row = jnp.sqrt(jnp.min(masked, axis=0, keepdims=True)) / rsc
    score = dmin_row * dens_row

    li = jax.lax.broadcasted_iota(i32, (1, n), 1)
    r16 = jax.lax.broadcasted_iota(i32, (cnp, n), 0)
    r16c = jax.lax.broadcasted_iota(i32, (cnp, 1), 0)
    onehot = jnp.zeros((cnp, n), dtype=f32)
    centerval = jnp.zeros((1, n), dtype=i32)
    iscenter = jnp.zeros((1, n), dtype=jnp.bool_)
    x2_sel = jnp.zeros((cnp, 1), dtype=f32)
    score_w = score
    for cc in range(cn):
        v = jnp.max(score_w, axis=1, keepdims=True)
        fi = jnp.min(jnp.where(score_w == v, li, n), axis=1, keepdims=True)
        sel = li == fi
        onehot = onehot + jnp.where((r16 == cc) & sel, 1.0, 0.0)
        centerval = jnp.where(sel, cc, centerval)
        iscenter = iscenter | sel
        x2c = jnp.sum(jnp.where(sel, x2_row, 0.0), axis=1, keepdims=True)
        x2_sel = jnp.where(r16c == cc, x2c, x2_sel)
        score_w = jnp.where(sel, -_BIG, score_w)

    ohb = onehot.astype(jnp.bfloat16)
    xcb = jax.lax.dot_general(ohb, xb, (((1,), (0,)), ((), ())),
                              preferred_element_type=f32).astype(jnp.bfloat16)
    g_sel = jax.lax.dot_general(xcb, xb, (((1,), (1,)), ((), ())),
                                preferred_element_type=f32)
    dm2 = jnp.maximum(x2_sel + x2_row - 2.0 * g_sel, 0.0)
    dm = jnp.sqrt(dm2) / rsc

    best = jnp.full((1, n), _BIG, dtype=f32)
    barg = jnp.zeros((1, n), dtype=i32)
    for cc in range(cn):
        row = jax.lax.slice(dm, (cc, 0), (cc + 1, n))
        upd = row < best
        best = jnp.where(upd, row, best)
        barg = jnp.where(upd, cc, barg)
    idx = jnp.where(iscenter, centerval, barg)

    wb = w_ref[:, :].astype(jnp.bfloat16)
    tscore = jax.lax.dot_general(wb, xb, (((1,), (1,)), ((), ())),
                                 preferred_element_type=f32)
    tw = jnp.exp(tscore + b_ref[:, :])
    a0 = (r16 == idx).astype(f32)
    p = a0 * tw
    allw = jnp.sum(p, axis=1, keepdims=True) + 1e-06
    a = p / allw
    merged = jax.lax.dot_general(a, x, (((1,), (0,)), ((), ())),
                                 preferred_element_type=f32)
    return jax.lax.slice(merged, (0, 0), (cn, x.shape[1]))


def kernel(patch_token, anomaly_map, W, b):
    del anomaly_map
    bsz, n, c = patch_token.shape
    cn = max(int(math.ceil(n * 0.01)), 1)
    cnp = ((cn + 7) // 8) * 8
    noise3 = jnp.asarray(_noise_const(bsz, n))
    b2 = jnp.reshape(b, (1, 1)).astype(jnp.float32)
    bb = 1

    def body(x_ref, noise_ref, w_ref, b_ref, out_ref):
        res = [_dpc_one(n, cn, cnp, x_ref[i], noise_ref[i, 0], w_ref, b_ref)
               for i in range(bb)]
        out_ref[:] = jnp.stack(res, axis=0)

    def call(x, nz, w, bbias):
        return pl.pallas_call(
            body,
            grid=(x.shape[0] // bb,),
            in_specs=[
                pl.BlockSpec((bb, n, c), lambda i: (i, 0, 0)),
                pl.BlockSpec((bb, 1, n), lambda i: (i, 0, 0)),
                pl.BlockSpec((1, c), lambda i: (0, 0)),
                pl.BlockSpec((1, 1), lambda i: (0, 0)),
            ],
            out_specs=pl.BlockSpec((bb, cn, c), lambda i: (i, 0, 0)),
            out_shape=jax.ShapeDtypeStruct((x.shape[0], cn, c), jnp.float32),
            compiler_params=pltpu.CompilerParams(
                dimension_semantics=("arbitrary",),
            ),
        )(x, nz, w, bbias)

    return call(patch_token, noise3, W, b2)


# speedup vs baseline: 4.8525x; 1.0010x over previous
import math

import jax
import jax.numpy as jnp
import numpy as np
from jax.experimental import pallas as pl
from jax.experimental.pallas import tpu as pltpu

_K = 5
_BIG = 1e30

_HI = jax.lax.Precision.HIGHEST

_NOISE_CACHE = {}


def _threefry2x32(k0, k1, x0, x1):
    def rol(x, d):
        return ((x << np.uint32(d)) | (x >> np.uint32(32 - d))).astype(np.uint32)

    ks2 = np.uint32(k0 ^ k1 ^ np.uint32(0x1BD11BDA))
    ks = [np.uint32(k0), np.uint32(k1), ks2]
    rot = ((13, 15, 26, 6), (17, 29, 16, 24))
    x0 = (x0 + ks[0]).astype(np.uint32)
    x1 = (x1 + ks[1]).astype(np.uint32)
    for i in range(5):
        for r in rot[i % 2]:
            x0 = (x0 + x1).astype(np.uint32)
            x1 = rol(x1, r) ^ x0
        x0 = (x0 + ks[(i + 1) % 3]).astype(np.uint32)
        x1 = (x1 + ks[(i + 2) % 3] + np.uint32(i + 1)).astype(np.uint32)
    return x0, x1


def _noise_const(bsz, n):
    key = (bsz, n)
    if key not in _NOISE_CACHE:
        cnt = bsz * n
        counts = np.arange(cnt, dtype=np.uint32)
        y0, y1 = _threefry2x32(np.uint32(0), np.uint32(42),
                               np.zeros(cnt, dtype=np.uint32), counts)
        bits = y0 ^ y1
        flt = ((bits >> np.uint32(9)) | np.uint32(0x3F800000)).view(np.float32)
        uni = np.maximum(np.float32(0.0), flt - np.float32(1.0))
        noise = (uni * np.float32(1e-06)).astype(np.float32)
        _NOISE_CACHE[key] = noise.reshape(bsz, 1, n)
    return _NOISE_CACHE[key]


def _dpc_one(n, cn, cnp, x, noise_row, w_ref, b_ref):
    f32 = jnp.float32
    i32 = jnp.int32
    c = x.shape[1]
    rsc = f32(math.sqrt(c))

    x2_col = jnp.sum(x * x, axis=1, keepdims=True)
    xb = x.astype(jnp.bfloat16)
    g = jax.lax.dot_general(xb, xb, (((1,), (1,)), ((), ())),
                            preferred_element_type=f32)
    x2_row = jnp.transpose(x2_col)
    d2 = jnp.maximum(x2_col + x2_row - 2.0 * g, 0.0)

    dw = d2
    acc = jnp.zeros((1, n), dtype=f32)
    rem = jnp.full((1, n), f32(_K))
    for t in range(_K):
        m = jnp.min(dw, axis=0, keepdims=True)
        eq = dw == m
        cnt = jnp.sum(jnp.where(eq, 1.0, 0.0), axis=0, keepdims=True)
        if t + 1 < _K:
            dw = jnp.where(eq, _BIG, dw)
        take = jnp.minimum(cnt, rem)
        rem = rem - take
        dn = jnp.sqrt(m) / rsc
        acc = acc + take * (dn * dn)
    dens_row = jnp.exp(-acc / f32(_K)) + noise_row
    dens_col = jnp.transpose(dens_row)

    d2max0 = jnp.max(d2, axis=0, keepdims=True)
    d2max = jnp.max(d2max0, axis=1, keepdims=True)
    masked = jnp.where(dens_col > dens_row, d2, d2max)
    dmin_row = jnp.sqrt(jnp.min(masked, axis=0, keepdims=True)) / rsc
    score = dmin_row * dens_row

    li = jax.lax.broadcasted_iota(i32, (1, n), 1)
    r16 = jax.lax.broadcasted_iota(i32, (cnp, n), 0)
    r16c = jax.lax.broadcasted_iota(i32, (cnp, 1), 0)
    onehot = jnp.zeros((cnp, n), dtype=f32)
    centerval = jnp.zeros((1, n), dtype=i32)
    iscenter = jnp.zeros((1, n), dtype=jnp.bool_)
    x2_sel = jnp.zeros((cnp, 1), dtype=f32)
    score_w = score
    for cc in range(cn):
        v = jnp.max(score_w, axis=1, keepdims=True)
        fi = jnp.min(jnp.where(score_w == v, li, n), axis=1, keepdims=True)
        sel = li == fi
        onehot = onehot + jnp.where((r16 == cc) & sel, 1.0, 0.0)
        centerval = jnp.where(sel, cc, centerval)
        iscenter = iscenter | sel
        x2c = jnp.sum(jnp.where(sel, x2_row, 0.0), axis=1, keepdims=True)
        x2_sel = jnp.where(r16c == cc, x2c, x2_sel)
        if cc + 1 < cn:
            score_w = jnp.where(sel, -_BIG, score_w)

    ohb = onehot.astype(jnp.bfloat16)
    xcb = jax.lax.dot_general(ohb, xb, (((1,), (0,)), ((), ())),
                              preferred_element_type=f32).astype(jnp.bfloat16)
    g_sel = jax.lax.dot_general(xcb, xb, (((1,), (1,)), ((), ())),
                                preferred_element_type=f32)
    dm2 = jnp.maximum(x2_sel + x2_row - 2.0 * g_sel, 0.0)
    dm = jnp.sqrt(dm2) / rsc

    best = jnp.full((1, n), _BIG, dtype=f32)
    barg = jnp.zeros((1, n), dtype=i32)
    for cc in range(cn):
        row = jax.lax.slice(dm, (cc, 0), (cc + 1, n))
        upd = row < best
        best = jnp.where(upd, row, best)
        barg = jnp.where(upd, cc, barg)
    idx = jnp.where(iscenter, centerval, barg)

    wb = w_ref[:, :].astype(jnp.bfloat16)
    tscore = jax.lax.dot_general(wb, xb, (((1,), (1,)), ((), ())),
                                 preferred_element_type=f32)
    tw = jnp.exp(tscore + b_ref[:, :])
    a0 = (r16 == idx).astype(f32)
    p = a0 * tw
    allw = jnp.sum(p, axis=1, keepdims=True) + 1e-06
    a = p / allw
    merged = jax.lax.dot_general(a, x, (((1,), (0,)), ((), ())),
                                 preferred_element_type=f32)
    return jax.lax.slice(merged, (0, 0), (cn, x.shape[1]))


def kernel(patch_token, anomaly_map, W, b):
    del anomaly_map
    bsz, n, c = patch_token.shape
    cn = max(int(math.ceil(n * 0.01)), 1)
    cnp = ((cn + 7) // 8) * 8
    noise3 = jnp.asarray(_noise_const(bsz, n))
    b2 = jnp.reshape(b, (1, 1)).astype(jnp.float32)
    bb = 1

    def body(x_ref, noise_ref, w_ref, b_ref, out_ref):
        res = [_dpc_one(n, cn, cnp, x_ref[i], noise_ref[i, 0], w_ref, b_ref)
               for i in range(bb)]
        out_ref[:] = jnp.stack(res, axis=0)

    def call(x, nz, w, bbias):
        return pl.pallas_call(
            body,
            grid=(x.shape[0] // bb,),
            in_specs=[
                pl.BlockSpec((bb, n, c), lambda i: (i, 0, 0)),
                pl.BlockSpec((bb, 1, n), lambda i: (i, 0, 0)),
                pl.BlockSpec((1, c), lambda i: (0, 0)),
                pl.BlockSpec((1, 1), lambda i: (0, 0)),
            ],
            out_specs=pl.BlockSpec((bb, cn, c), lambda i: (i, 0, 0)),
            out_shape=jax.ShapeDtypeStruct((x.shape[0], cn, c), jnp.float32),
            compiler_params=pltpu.CompilerParams(
                dimension_semantics=("arbitrary",),
            ),
        )(x, nz, w, bbias)

    return call(patch_token, noise3, W, b2)
